# Initial kernel scaffold; baseline (speedup 1.0000x reference)
import jax
import jax.numpy as jnp
from jax.experimental import pallas as pl


def kernel(relative_position_bias_table, learnable_offset):
    raise NotImplementedError("write your pallas kernel here")


# speedup vs baseline: 82.9627x; 82.9627x over previous
import functools

import jax
import jax.numpy as jnp
from jax import lax
from jax.experimental import pallas as pl
from jax.experimental.pallas import tpu as pltpu
from jax.experimental.pallas import tpu_sc as plsc

T = 2048
H = 16
L = 16
PAD = 16
NCHUNK = (2 * T) // L
GROUPS_PER_HALF = T // 8 // 2
ROWSTRIDE = 2 * T + PAD


def _sc_body(tr_hbm, off_hbm, out_hbm, tr_v, s8_v, off_v, sem):
    c = lax.axis_index("c")
    s = lax.axis_index("s")
    head = s

    pltpu.sync_copy(tr_hbm.at[pl.ds(head * (2 * T), 2 * T)],
                    tr_v.at[pl.ds(PAD, 2 * T)])
    pltpu.sync_copy(off_hbm, off_v)

    x = off_v[...]
    e = jnp.exp(x + x)
    bo = (1.0 - 2.0 / (e + 1.0)) * 0.5
    pos = bo >= 0.0

    def chunk(i, carry):
        k0 = i * L
        kk = lax.iota(jnp.int32, L) + k0
        d = 4094.0 - kk.astype(jnp.float32)
        adj = jnp.clip(d + bo, 0.0, 4094.0)
        fl = adj.astype(jnp.int32).astype(jnp.float32)
        w = adj - fl
        t0 = tr_v[pl.ds(PAD + k0, L)]
        tl = tr_v[pl.ds(PAD + k0 + 1, L)]
        tm = tr_v[pl.ds(PAD + k0 - 1, L)]
        a = jnp.where(pos, t0, tl)
        b = jnp.where(pos, tm, t0)
        v = a * (1.0 - w) + b * w
        for r in range(8):
            s8_v[pl.ds(r * ROWSTRIDE + PAD + k0 + r - 7, L)] = v
        return carry

    lax.fori_loop(0, NCHUNK, chunk, 0)

    g0 = c * GROUPS_PER_HALF

    row_base = head * (T * T)

    def fire(j, carry):
        g = g0 + j
        start = PAD + 2040 - 8 * g
        for r in range(8):
            pltpu.async_copy(
                s8_v.at[pl.ds(r * ROWSTRIDE + start, T)],
                out_hbm.at[pl.ds(row_base + (8 * g + r) * T, T)],
                sem,
            )
        return carry

    lax.fori_loop(0, GROUPS_PER_HALF, fire, 0)

    def drain(j, carry):
        g = g0 + j
        start = PAD + 2040 - 8 * g
        for r in range(8):
            pltpu.make_async_copy(
                s8_v.at[pl.ds(r * ROWSTRIDE + start, T)],
                out_hbm.at[pl.ds(row_base + (8 * g + r) * T, T)],
                sem,
            ).wait()
        return carry

    lax.fori_loop(0, GROUPS_PER_HALF, drain, 0)


_sc_call = functools.partial(
    pl.kernel,
    out_type=jax.ShapeDtypeStruct((H * T * T,), jnp.float32),
    mesh=plsc.VectorSubcoreMesh(core_axis_name="c", subcore_axis_name="s"),
    scratch_types=[
        pltpu.VMEM((2 * T + 2 * PAD, ), jnp.float32),
        pltpu.VMEM((8 * ROWSTRIDE,), jnp.float32),
        pltpu.VMEM((L,), jnp.float32),
        pltpu.SemaphoreType.DMA,
    ],
)(_sc_body)


def kernel(relative_position_bias_table, learnable_offset):
    tbl = relative_position_bias_table
    trp = jnp.concatenate([tbl[::-1], tbl[:1]], axis=0).T
    off16 = jnp.broadcast_to(learnable_offset.astype(jnp.float32), (L,))
    flat = _sc_call(jnp.asarray(trp, jnp.float32).reshape(-1), off16)
    return flat.reshape(1, H, T, T)
